# baseline (speedup 1.0000x reference)
import functools

import jax
import jax.numpy as jnp
from jax import lax
from jax.experimental import pallas as pl
from jax.experimental.pallas import tpu as pltpu
from jax.experimental.pallas import tpu_sc as plsc

N_NODES = 10000
N_EDGES = 320000
DIM = 128
BN_EPS = 1e-5

NC, NS, L = 2, 16, 16
NW = NC * NS
CHUNK = 128
BATCH = 2
N_PAD = 10240
ROWS_PER_TILE = N_PAD // NS
E_PAD = 327680
K0, K1 = 98, 62
DEG_CHUNKS = E_PAD // (NW * CHUNK)
DEG_BATCH = 8

_mesh = plsc.VectorSubcoreMesh(
    core_axis_name="c", subcore_axis_name="s", num_cores=NC, num_subcores=NS)


@functools.partial(
    pl.kernel,
    out_type=jax.ShapeDtypeStruct((NC * N_PAD,), jnp.float32),
    mesh=_mesh,
    scratch_types=[pltpu.VMEM((CHUNK,), jnp.int32)] * DEG_BATCH + [
        pltpu.VMEM((CHUNK,), jnp.float32),
        pltpu.VMEM((CHUNK,), jnp.float32),
        pltpu.VMEM_SHARED((N_PAD,), jnp.float32),
    ] + [pltpu.SemaphoreType.DMA] * (2 * DEG_BATCH),
)
def _deg_kernel(dst_hbm, deg_out, *rest):
    didx = rest[:DEG_BATCH]
    ones_v, zbuf_v, deg_sh = rest[DEG_BATCH:DEG_BATCH + 3]
    semi = rest[DEG_BATCH + 3:2 * DEG_BATCH + 3]
    sems = rest[2 * DEG_BATCH + 3:]
    cid = lax.axis_index("c")
    sid = lax.axis_index("s")
    wid = cid * NS + sid
    for i in range(CHUNK // L):
        ones_v[pl.ds(i * L, L)] = jnp.full((L,), 1.0, jnp.float32)
        zbuf_v[pl.ds(i * L, L)] = jnp.zeros((L,), jnp.float32)
    r0 = pl.multiple_of(sid * ROWS_PER_TILE, 128)
    for k in range(ROWS_PER_TILE // CHUNK):
        pltpu.sync_copy(zbuf_v, deg_sh.at[pl.ds(r0 + k * CHUNK, CHUNK)])
    plsc.subcore_barrier()
    base = wid * DEG_CHUNKS * CHUNK

    def body(t, _):
        c0 = base + t * DEG_BATCH * CHUNK
        ih = [
            pltpu.async_copy(dst_hbm.at[pl.ds(c0 + b * CHUNK, CHUNK)],
                             didx[b], semi[b])
            for b in range(DEG_BATCH)
        ]
        sh = []
        for b in range(DEG_BATCH):
            ih[b].wait()
            sh.append(pltpu.async_copy(ones_v, deg_sh.at[didx[b]], sems[b],
                                       add=True))
        for h in sh:
            h.wait()
        return ()

    lax.fori_loop(0, DEG_CHUNKS // DEG_BATCH, body, (), unroll=False)
    plsc.subcore_barrier()
    o0 = pl.multiple_of(cid * N_PAD + r0, 128)
    pltpu.sync_copy(deg_sh.at[pl.ds(r0, ROWS_PER_TILE)],
                    deg_out.at[pl.ds(o0, ROWS_PER_TILE)])


@functools.partial(
    pl.kernel,
    out_type=jax.ShapeDtypeStruct((NC * N_PAD, DIM), jnp.float32),
    mesh=_mesh,
    scratch_types=[pltpu.VMEM((CHUNK,), jnp.int32)] * (2 * BATCH) + [
        pltpu.VMEM((BATCH, CHUNK, DIM), jnp.float32),
        pltpu.VMEM_SHARED((N_PAD, DIM), jnp.float32),
    ] + [pltpu.SemaphoreType.DMA] * (4 * BATCH),
)
def _gather_scatter_kernel(xs_hbm, src_hbm, dst_hbm, acc_out, *rest):
    sidx = rest[:BATCH]
    didx = rest[BATCH:2 * BATCH]
    rows_v, acc_sh = rest[2 * BATCH:2 * BATCH + 2]
    sems = rest[2 * BATCH + 2:]
    semi = sems[:BATCH]
    semi2 = sems[BATCH:2 * BATCH]
    semg = sems[2 * BATCH:3 * BATCH]
    semsc = sems[3 * BATCH:]
    cid = lax.axis_index("c")
    sid = lax.axis_index("s")
    r0 = pl.multiple_of(sid * ROWS_PER_TILE, 128)

    def zrow(r, _):
        for i in range(DIM // L):
            rows_v[0, r, pl.ds(i * L, L)] = jnp.zeros((L,), jnp.float32)
        return ()

    lax.fori_loop(0, CHUNK, zrow, (), unroll=False)
    for k in range(ROWS_PER_TILE // CHUNK):
        pltpu.sync_copy(rows_v.at[0], acc_sh.at[pl.ds(r0 + k * CHUNK, CHUNK)])
    plsc.subcore_barrier()

    nk = jnp.where(cid == 0, K0, K1)
    base = jnp.where(cid == 0, sid * K0, NS * K0 + sid * K1) * CHUNK

    def body(t2, _):
        for b in range(BATCH):
            c = t2 * BATCH + b
            off = base + c * CHUNK

            @pl.when(c >= BATCH)
            def _():
                pltpu.make_async_copy(rows_v.at[b], acc_sh.at[didx[b]],
                                      semsc[b]).wait()

            h1 = pltpu.async_copy(src_hbm.at[pl.ds(off, CHUNK)], sidx[b],
                                  semi[b])
            h2 = pltpu.async_copy(dst_hbm.at[pl.ds(off, CHUNK)], didx[b],
                                  semi2[b])
            h1.wait()
            pltpu.async_copy(xs_hbm.at[sidx[b]], rows_v.at[b], semg[b]).wait()
            h2.wait()
            pltpu.async_copy(rows_v.at[b], acc_sh.at[didx[b]], semsc[b],
                             add=True)
        return ()

    lax.fori_loop(0, nk // BATCH, body, (), unroll=False)
    for b in range(BATCH):
        pltpu.make_async_copy(rows_v.at[b], acc_sh.at[didx[b]],
                              semsc[b]).wait()

    plsc.subcore_barrier()
    o0 = pl.multiple_of(cid * N_PAD + r0, 128)
    WB = 4 * BATCH
    wrows = ROWS_PER_TILE // WB
    wh = [
        pltpu.async_copy(acc_sh.at[pl.ds(r0 + j * wrows, wrows)],
                         acc_out.at[pl.ds(o0 + j * wrows, wrows)], sems[j])
        for j in range(WB)
    ]
    for h in wh:
        h.wait()


ROW_BLK = 1000


def _scale_body(x_ref, w_ref, d0_ref, d1_ref, xs_ref):
    dinv = lax.rsqrt(d0_ref[...] + d1_ref[...] + 1.0)
    xw = jnp.dot(x_ref[...], w_ref[...], preferred_element_type=jnp.float32)
    xs_ref[...] = xw * dinv


def _scale_kernel(x, w, d0, d1):
    return pl.pallas_call(
        _scale_body,
        out_shape=jax.ShapeDtypeStruct((N_NODES, DIM), jnp.float32),
        grid=(N_NODES // ROW_BLK,),
        in_specs=[
            pl.BlockSpec((ROW_BLK, DIM), lambda i: (i, 0)),
            pl.BlockSpec((DIM, DIM), lambda i: (0, 0)),
            pl.BlockSpec((ROW_BLK, 1), lambda i: (i, 0)),
            pl.BlockSpec((ROW_BLK, 1), lambda i: (i, 0)),
        ],
        out_specs=pl.BlockSpec((ROW_BLK, DIM), lambda i: (i, 0)),
    )(x, w, d0, d1)


def _final_body(x_ref, xs_ref, a0_ref, a1_ref, d0_ref, d1_ref, bg_ref,
                g1_ref, be1_ref, w1_ref, b1_ref, w2_ref, b2_ref, g2_ref,
                be2_ref, out_ref):
    c = 1.0 / (1.0 + BN_EPS) ** 0.5
    dinv = lax.rsqrt(d0_ref[...] + d1_ref[...] + 1.0)
    acc = a0_ref[...] + a1_ref[...] + xs_ref[...]
    h = x_ref[...] + dinv * acc + bg_ref[...]
    h = g1_ref[...] * (h * c) + be1_ref[...]
    t = jnp.dot(h, w1_ref[...], preferred_element_type=jnp.float32)
    t = jnp.maximum(t + b1_ref[...], 0.0)
    ff = jnp.dot(t, w2_ref[...], preferred_element_type=jnp.float32)
    h = h + ff + b2_ref[...]
    out_ref[...] = g2_ref[...] * (h * c) + be2_ref[...]


def _final_kernel(x, xs, a0, a1, d0, d1, b_gcn, g1, be1, w1, b1, w2, b2,
                  g2, be2):
    row = lambda i: (i, 0)
    full = lambda shape: pl.BlockSpec(shape, lambda i: (0, 0))
    return pl.pallas_call(
        _final_body,
        out_shape=jax.ShapeDtypeStruct((N_NODES, DIM), jnp.float32),
        grid=(N_NODES // ROW_BLK,),
        in_specs=[
            pl.BlockSpec((ROW_BLK, DIM), row),
            pl.BlockSpec((ROW_BLK, DIM), row),
            pl.BlockSpec((ROW_BLK, DIM), row),
            pl.BlockSpec((ROW_BLK, DIM), row),
            pl.BlockSpec((ROW_BLK, 1), row),
            pl.BlockSpec((ROW_BLK, 1), row),
            full((1, DIM)),
            full((1, DIM)),
            full((1, DIM)),
            full((DIM, 2 * DIM)),
            full((1, 2 * DIM)),
            full((2 * DIM, DIM)),
            full((1, DIM)),
            full((1, DIM)),
            full((1, DIM)),
        ],
        out_specs=pl.BlockSpec((ROW_BLK, DIM), row),
    )(x, xs, a0, a1, d0, d1, b_gcn, g1, be1, w1, b1, w2, b2, g2, be2)


def kernel(x, edge_index, edge_attr, W_gcn, b_gcn, bn1_g, bn1_b, W1, b1,
           W2, b2, bn2_g, bn2_b):
    del edge_attr
    src = edge_index[0].astype(jnp.int32)
    dst = edge_index[1].astype(jnp.int32)
    npad = E_PAD - N_EDGES
    src = jnp.concatenate([src, jnp.zeros((npad,), jnp.int32)])
    dst = jnp.concatenate([dst, jnp.full((npad,), N_NODES, jnp.int32)])

    deg = _deg_kernel(dst).reshape(NC, N_PAD)
    d0 = deg[0, :N_NODES, None]
    d1 = deg[1, :N_NODES, None]
    xs = _scale_kernel(x, W_gcn, d0, d1)
    acc = _gather_scatter_kernel(xs, src, dst).reshape(NC, N_PAD, DIM)
    return _final_kernel(
        x, xs, acc[0, :N_NODES], acc[1, :N_NODES], d0, d1,
        b_gcn[None, :], bn1_g[None, :], bn1_b[None, :],
        W1, b1[None, :], W2, b2[None, :], bn2_g[None, :], bn2_b[None, :])


# speedup vs baseline: 1.4374x; 1.4374x over previous
import functools

import jax
import jax.numpy as jnp
from jax import lax
from jax.experimental import pallas as pl
from jax.experimental.pallas import tpu as pltpu
from jax.experimental.pallas import tpu_sc as plsc

N_NODES = 10000
N_EDGES = 320000
DIM = 128
BN_EPS = 1e-5

NC, NS, L = 2, 16, 16
NW = NC * NS
CHUNK = 128
BATCH = 2
N_PAD = 10240
ROWS_PER_TILE = N_PAD // NS
E_PAD = 327680
K0, K1 = 144, 16
DEG_CHUNKS = E_PAD // (NW * CHUNK)
DEG_BATCH = 8

_mesh = plsc.VectorSubcoreMesh(
    core_axis_name="c", subcore_axis_name="s", num_cores=NC, num_subcores=NS)


@functools.partial(
    pl.kernel,
    out_type=jax.ShapeDtypeStruct((NC * N_PAD,), jnp.float32),
    mesh=_mesh,
    scratch_types=[pltpu.VMEM((CHUNK,), jnp.int32)] * DEG_BATCH + [
        pltpu.VMEM((CHUNK,), jnp.float32),
        pltpu.VMEM((CHUNK,), jnp.float32),
        pltpu.VMEM_SHARED((N_PAD,), jnp.float32),
    ] + [pltpu.SemaphoreType.DMA] * (2 * DEG_BATCH),
)
def _deg_kernel(dst_hbm, deg_out, *rest):
    didx = rest[:DEG_BATCH]
    ones_v, zbuf_v, deg_sh = rest[DEG_BATCH:DEG_BATCH + 3]
    semi = rest[DEG_BATCH + 3:2 * DEG_BATCH + 3]
    sems = rest[2 * DEG_BATCH + 3:]
    cid = lax.axis_index("c")
    sid = lax.axis_index("s")
    wid = cid * NS + sid
    for i in range(CHUNK // L):
        ones_v[pl.ds(i * L, L)] = jnp.full((L,), 1.0, jnp.float32)
        zbuf_v[pl.ds(i * L, L)] = jnp.zeros((L,), jnp.float32)
    r0 = pl.multiple_of(sid * ROWS_PER_TILE, 128)
    for k in range(ROWS_PER_TILE // CHUNK):
        pltpu.sync_copy(zbuf_v, deg_sh.at[pl.ds(r0 + k * CHUNK, CHUNK)])
    plsc.subcore_barrier()
    base = wid * DEG_CHUNKS * CHUNK

    def body(t, _):
        c0 = base + t * DEG_BATCH * CHUNK
        ih = [
            pltpu.async_copy(dst_hbm.at[pl.ds(c0 + b * CHUNK, CHUNK)],
                             didx[b], semi[b])
            for b in range(DEG_BATCH)
        ]
        sh = []
        for b in range(DEG_BATCH):
            ih[b].wait()
            sh.append(pltpu.async_copy(ones_v, deg_sh.at[didx[b]], sems[b],
                                       add=True))
        for h in sh:
            h.wait()
        return ()

    lax.fori_loop(0, DEG_CHUNKS // DEG_BATCH, body, (), unroll=False)
    plsc.subcore_barrier()
    o0 = pl.multiple_of(cid * N_PAD + r0, 128)
    pltpu.sync_copy(deg_sh.at[pl.ds(r0, ROWS_PER_TILE)],
                    deg_out.at[pl.ds(o0, ROWS_PER_TILE)])


@functools.partial(
    pl.kernel,
    out_type=jax.ShapeDtypeStruct((NC * N_PAD, DIM), jnp.float32),
    mesh=_mesh,
    scratch_types=[pltpu.VMEM((CHUNK,), jnp.int32)] * (2 * BATCH) + [
        pltpu.VMEM((BATCH, CHUNK, DIM), jnp.float32),
        pltpu.VMEM_SHARED((N_PAD, DIM), jnp.float32),
    ] + [pltpu.SemaphoreType.DMA] * (4 * BATCH),
)
def _gather_scatter_kernel(xs_hbm, src_hbm, dst_hbm, acc_out, *rest):
    sidx = rest[:BATCH]
    didx = rest[BATCH:2 * BATCH]
    rows_v, acc_sh = rest[2 * BATCH:2 * BATCH + 2]
    sems = rest[2 * BATCH + 2:]
    semi = sems[:BATCH]
    semi2 = sems[BATCH:2 * BATCH]
    semg = sems[2 * BATCH:3 * BATCH]
    semsc = sems[3 * BATCH:]
    cid = lax.axis_index("c")
    sid = lax.axis_index("s")
    r0 = pl.multiple_of(sid * ROWS_PER_TILE, 128)

    def zrow(r, _):
        for i in range(DIM // L):
            rows_v[0, r, pl.ds(i * L, L)] = jnp.zeros((L,), jnp.float32)
        return ()

    lax.fori_loop(0, CHUNK, zrow, (), unroll=False)
    for k in range(ROWS_PER_TILE // CHUNK):
        pltpu.sync_copy(rows_v.at[0], acc_sh.at[pl.ds(r0 + k * CHUNK, CHUNK)])
    plsc.subcore_barrier()

    @pl.when(cid == 0)
    def _():
        base = sid * K0 * CHUNK

        def body(t, _):
            c0 = base + t * BATCH * CHUNK
            ih = []
            for b in range(BATCH):
                off = c0 + b * CHUNK
                ih.append((
                    pltpu.async_copy(src_hbm.at[pl.ds(off, CHUNK)], sidx[b],
                                     semi[b]),
                    pltpu.async_copy(dst_hbm.at[pl.ds(off, CHUNK)], didx[b],
                                     semi2[b]),
                ))
            gh = []
            for b in range(BATCH):
                ih[b][0].wait()
                gh.append(pltpu.async_copy(xs_hbm.at[sidx[b]], rows_v.at[b],
                                           semg[b]))
            sh = []
            for b in range(BATCH):
                gh[b].wait()
                ih[b][1].wait()
                sh.append(pltpu.async_copy(rows_v.at[b], acc_sh.at[didx[b]],
                                           semsc[b], add=True))
            for h in sh:
                h.wait()
            return ()

        lax.fori_loop(0, K0 // BATCH, body, (), unroll=False)

    @pl.when(cid == 1)
    def _():
        base = (NS * K0 + sid * K1) * CHUNK

        def body(t, _):
            off = base + t * CHUNK
            h1 = pltpu.async_copy(src_hbm.at[pl.ds(off, CHUNK)], sidx[0],
                                  semi[0])
            h2 = pltpu.async_copy(dst_hbm.at[pl.ds(off, CHUNK)], didx[0],
                                  semi2[0])
            h1.wait()
            pltpu.async_copy(xs_hbm.at[sidx[0]], rows_v.at[0], semg[0]).wait()
            h2.wait()
            pltpu.async_copy(rows_v.at[0], acc_sh.at[didx[0]], semsc[0],
                             add=True).wait()
            return ()

        lax.fori_loop(0, K1, body, (), unroll=False)

    plsc.subcore_barrier()
    o0 = pl.multiple_of(cid * N_PAD + r0, 128)
    WB = 4 * BATCH
    wrows = ROWS_PER_TILE // WB
    wh = [
        pltpu.async_copy(acc_sh.at[pl.ds(r0 + j * wrows, wrows)],
                         acc_out.at[pl.ds(o0 + j * wrows, wrows)], sems[j])
        for j in range(WB)
    ]
    for h in wh:
        h.wait()


ROW_BLK = 1000


def _scale_body(x_ref, w_ref, d0_ref, d1_ref, xs_ref):
    dinv = lax.rsqrt(d0_ref[...] + d1_ref[...] + 1.0)
    xw = jnp.dot(x_ref[...], w_ref[...], preferred_element_type=jnp.float32)
    xs_ref[...] = xw * dinv


def _scale_kernel(x, w, d0, d1):
    return pl.pallas_call(
        _scale_body,
        out_shape=jax.ShapeDtypeStruct((N_NODES, DIM), jnp.float32),
        grid=(N_NODES // ROW_BLK,),
        in_specs=[
            pl.BlockSpec((ROW_BLK, DIM), lambda i: (i, 0)),
            pl.BlockSpec((DIM, DIM), lambda i: (0, 0)),
            pl.BlockSpec((ROW_BLK, 1), lambda i: (i, 0)),
            pl.BlockSpec((ROW_BLK, 1), lambda i: (i, 0)),
        ],
        out_specs=pl.BlockSpec((ROW_BLK, DIM), lambda i: (i, 0)),
    )(x, w, d0, d1)


def _final_body(x_ref, xs_ref, a0_ref, a1_ref, d0_ref, d1_ref, bg_ref,
                g1_ref, be1_ref, w1_ref, b1_ref, w2_ref, b2_ref, g2_ref,
                be2_ref, out_ref):
    c = 1.0 / (1.0 + BN_EPS) ** 0.5
    dinv = lax.rsqrt(d0_ref[...] + d1_ref[...] + 1.0)
    acc = a0_ref[...] + a1_ref[...] + xs_ref[...]
    h = x_ref[...] + dinv * acc + bg_ref[...]
    h = g1_ref[...] * (h * c) + be1_ref[...]
    t = jnp.dot(h, w1_ref[...], preferred_element_type=jnp.float32)
    t = jnp.maximum(t + b1_ref[...], 0.0)
    ff = jnp.dot(t, w2_ref[...], preferred_element_type=jnp.float32)
    h = h + ff + b2_ref[...]
    out_ref[...] = g2_ref[...] * (h * c) + be2_ref[...]


def _final_kernel(x, xs, a0, a1, d0, d1, b_gcn, g1, be1, w1, b1, w2, b2,
                  g2, be2):
    row = lambda i: (i, 0)
    full = lambda shape: pl.BlockSpec(shape, lambda i: (0, 0))
    return pl.pallas_call(
        _final_body,
        out_shape=jax.ShapeDtypeStruct((N_NODES, DIM), jnp.float32),
        grid=(N_NODES // ROW_BLK,),
        in_specs=[
            pl.BlockSpec((ROW_BLK, DIM), row),
            pl.BlockSpec((ROW_BLK, DIM), row),
            pl.BlockSpec((ROW_BLK, DIM), row),
            pl.BlockSpec((ROW_BLK, DIM), row),
            pl.BlockSpec((ROW_BLK, 1), row),
            pl.BlockSpec((ROW_BLK, 1), row),
            full((1, DIM)),
            full((1, DIM)),
            full((1, DIM)),
            full((DIM, 2 * DIM)),
            full((1, 2 * DIM)),
            full((2 * DIM, DIM)),
            full((1, DIM)),
            full((1, DIM)),
            full((1, DIM)),
        ],
        out_specs=pl.BlockSpec((ROW_BLK, DIM), row),
    )(x, xs, a0, a1, d0, d1, b_gcn, g1, be1, w1, b1, w2, b2, g2, be2)


def kernel(x, edge_index, edge_attr, W_gcn, b_gcn, bn1_g, bn1_b, W1, b1,
           W2, b2, bn2_g, bn2_b):
    del edge_attr
    src = edge_index[0].astype(jnp.int32)
    dst = edge_index[1].astype(jnp.int32)
    npad = E_PAD - N_EDGES
    src = jnp.concatenate([src, jnp.zeros((npad,), jnp.int32)])
    dst = jnp.concatenate([dst, jnp.full((npad,), N_NODES, jnp.int32)])

    deg = _deg_kernel(dst).reshape(NC, N_PAD)
    d0 = deg[0, :N_NODES, None]
    d1 = deg[1, :N_NODES, None]
    xs = _scale_kernel(x, W_gcn, d0, d1)
    acc = _gather_scatter_kernel(xs, src, dst).reshape(NC, N_PAD, DIM)
    return _final_kernel(
        x, xs, acc[0, :N_NODES], acc[1, :N_NODES], d0, d1,
        b_gcn[None, :], bn1_g[None, :], bn1_b[None, :],
        W1, b1[None, :], W2, b2[None, :], bn2_g[None, :], bn2_b[None, :])


# speedup vs baseline: 1.4686x; 1.0217x over previous
import functools

import jax
import jax.numpy as jnp
from jax import lax
from jax.experimental import pallas as pl
from jax.experimental.pallas import tpu as pltpu
from jax.experimental.pallas import tpu_sc as plsc

N_NODES = 10000
N_EDGES = 320000
DIM = 128
BN_EPS = 1e-5

NC, NS, L = 2, 16, 16
NW = NC * NS
CHUNK = 128
BATCH = 2
N_PAD = 10240
ROWS_PER_TILE = N_PAD // NS
E_PAD = 327680
K0, K1 = 152, 8
DEG_CHUNKS = E_PAD // (NW * CHUNK)
DEG_BATCH = 8

_mesh = plsc.VectorSubcoreMesh(
    core_axis_name="c", subcore_axis_name="s", num_cores=NC, num_subcores=NS)


@functools.partial(
    pl.kernel,
    out_type=jax.ShapeDtypeStruct((NC * N_PAD,), jnp.float32),
    mesh=_mesh,
    scratch_types=[pltpu.VMEM((CHUNK,), jnp.int32)] * DEG_BATCH + [
        pltpu.VMEM((CHUNK,), jnp.float32),
        pltpu.VMEM((CHUNK,), jnp.float32),
        pltpu.VMEM_SHARED((N_PAD,), jnp.float32),
    ] + [pltpu.SemaphoreType.DMA] * (2 * DEG_BATCH),
)
def _deg_kernel(dst_hbm, deg_out, *rest):
    didx = rest[:DEG_BATCH]
    ones_v, zbuf_v, deg_sh = rest[DEG_BATCH:DEG_BATCH + 3]
    semi = rest[DEG_BATCH + 3:2 * DEG_BATCH + 3]
    sems = rest[2 * DEG_BATCH + 3:]
    cid = lax.axis_index("c")
    sid = lax.axis_index("s")
    wid = cid * NS + sid
    for i in range(CHUNK // L):
        ones_v[pl.ds(i * L, L)] = jnp.full((L,), 1.0, jnp.float32)
        zbuf_v[pl.ds(i * L, L)] = jnp.zeros((L,), jnp.float32)
    r0 = pl.multiple_of(sid * ROWS_PER_TILE, 128)
    for k in range(ROWS_PER_TILE // CHUNK):
        pltpu.sync_copy(zbuf_v, deg_sh.at[pl.ds(r0 + k * CHUNK, CHUNK)])
    plsc.subcore_barrier()
    base = wid * DEG_CHUNKS * CHUNK

    def body(t, _):
        c0 = base + t * DEG_BATCH * CHUNK
        ih = [
            pltpu.async_copy(dst_hbm.at[pl.ds(c0 + b * CHUNK, CHUNK)],
                             didx[b], semi[b])
            for b in range(DEG_BATCH)
        ]
        sh = []
        for b in range(DEG_BATCH):
            ih[b].wait()
            sh.append(pltpu.async_copy(ones_v, deg_sh.at[didx[b]], sems[b],
                                       add=True))
        for h in sh:
            h.wait()
        return ()

    lax.fori_loop(0, DEG_CHUNKS // DEG_BATCH, body, (), unroll=False)
    plsc.subcore_barrier()
    o0 = pl.multiple_of(cid * N_PAD + r0, 128)
    pltpu.sync_copy(deg_sh.at[pl.ds(r0, ROWS_PER_TILE)],
                    deg_out.at[pl.ds(o0, ROWS_PER_TILE)])


@functools.partial(
    pl.kernel,
    out_type=jax.ShapeDtypeStruct((NC * N_PAD, DIM), jnp.float32),
    mesh=_mesh,
    scratch_types=[pltpu.VMEM((CHUNK,), jnp.int32)] * (2 * BATCH) + [
        pltpu.VMEM((BATCH, CHUNK, DIM), jnp.float32),
        pltpu.VMEM_SHARED((N_PAD, DIM), jnp.float32),
    ] + [pltpu.SemaphoreType.DMA] * (4 * BATCH),
)
def _gather_scatter_kernel(xs_hbm, src_hbm, dst_hbm, acc_out, *rest):
    sidx = rest[:BATCH]
    didx = rest[BATCH:2 * BATCH]
    rows_v, acc_sh = rest[2 * BATCH:2 * BATCH + 2]
    sems = rest[2 * BATCH + 2:]
    semi = sems[:BATCH]
    semi2 = sems[BATCH:2 * BATCH]
    semg = sems[2 * BATCH:3 * BATCH]
    semsc = sems[3 * BATCH:]
    cid = lax.axis_index("c")
    sid = lax.axis_index("s")
    r0 = pl.multiple_of(sid * ROWS_PER_TILE, 128)

    def zrow(r, _):
        for i in range(DIM // L):
            rows_v[0, r, pl.ds(i * L, L)] = jnp.zeros((L,), jnp.float32)
        return ()

    lax.fori_loop(0, CHUNK, zrow, (), unroll=False)
    for k in range(ROWS_PER_TILE // CHUNK):
        pltpu.sync_copy(rows_v.at[0], acc_sh.at[pl.ds(r0 + k * CHUNK, CHUNK)])
    plsc.subcore_barrier()

    @pl.when(cid == 0)
    def _():
        base = sid * K0 * CHUNK

        def body(t, _):
            c0 = base + t * BATCH * CHUNK
            ih = []
            for b in range(BATCH):
                off = c0 + b * CHUNK
                ih.append((
                    pltpu.async_copy(src_hbm.at[pl.ds(off, CHUNK)], sidx[b],
                                     semi[b]),
                    pltpu.async_copy(dst_hbm.at[pl.ds(off, CHUNK)], didx[b],
                                     semi2[b]),
                ))
            gh = []
            for b in range(BATCH):
                ih[b][0].wait()
                gh.append(pltpu.async_copy(xs_hbm.at[sidx[b]], rows_v.at[b],
                                           semg[b]))
            sh = []
            for b in range(BATCH):
                gh[b].wait()
                ih[b][1].wait()
                sh.append(pltpu.async_copy(rows_v.at[b], acc_sh.at[didx[b]],
                                           semsc[b], add=True))
            for h in sh:
                h.wait()
            return ()

        lax.fori_loop(0, K0 // BATCH, body, (), unroll=False)

    @pl.when(cid == 1)
    def _():
        base = (NS * K0 + sid * K1) * CHUNK

        def body(t, _):
            off = base + t * CHUNK
            h1 = pltpu.async_copy(src_hbm.at[pl.ds(off, CHUNK)], sidx[0],
                                  semi[0])
            h2 = pltpu.async_copy(dst_hbm.at[pl.ds(off, CHUNK)], didx[0],
                                  semi2[0])
            h1.wait()
            pltpu.async_copy(xs_hbm.at[sidx[0]], rows_v.at[0], semg[0]).wait()
            h2.wait()
            pltpu.async_copy(rows_v.at[0], acc_sh.at[didx[0]], semsc[0],
                             add=True).wait()
            return ()

        lax.fori_loop(0, K1, body, (), unroll=False)

    plsc.subcore_barrier()
    o0 = pl.multiple_of(cid * N_PAD + r0, 128)
    WB = 4 * BATCH
    wrows = ROWS_PER_TILE // WB
    wh = [
        pltpu.async_copy(acc_sh.at[pl.ds(r0 + j * wrows, wrows)],
                         acc_out.at[pl.ds(o0 + j * wrows, wrows)], sems[j])
        for j in range(WB)
    ]
    for h in wh:
        h.wait()


ROW_BLK = 1000


def _scale_body(x_ref, w_ref, d0_ref, d1_ref, xs_ref):
    dinv = lax.rsqrt(d0_ref[...] + d1_ref[...] + 1.0)
    xw = jnp.dot(x_ref[...], w_ref[...], preferred_element_type=jnp.float32)
    xs_ref[...] = xw * dinv


def _scale_kernel(x, w, d0, d1):
    return pl.pallas_call(
        _scale_body,
        out_shape=jax.ShapeDtypeStruct((N_NODES, DIM), jnp.float32),
        grid=(N_NODES // ROW_BLK,),
        in_specs=[
            pl.BlockSpec((ROW_BLK, DIM), lambda i: (i, 0)),
            pl.BlockSpec((DIM, DIM), lambda i: (0, 0)),
            pl.BlockSpec((ROW_BLK, 1), lambda i: (i, 0)),
            pl.BlockSpec((ROW_BLK, 1), lambda i: (i, 0)),
        ],
        out_specs=pl.BlockSpec((ROW_BLK, DIM), lambda i: (i, 0)),
    )(x, w, d0, d1)


def _final_body(x_ref, xs_ref, a0_ref, a1_ref, d0_ref, d1_ref, bg_ref,
                g1_ref, be1_ref, w1_ref, b1_ref, w2_ref, b2_ref, g2_ref,
                be2_ref, out_ref):
    c = 1.0 / (1.0 + BN_EPS) ** 0.5
    dinv = lax.rsqrt(d0_ref[...] + d1_ref[...] + 1.0)
    acc = a0_ref[...] + a1_ref[...] + xs_ref[...]
    h = x_ref[...] + dinv * acc + bg_ref[...]
    h = g1_ref[...] * (h * c) + be1_ref[...]
    t = jnp.dot(h, w1_ref[...], preferred_element_type=jnp.float32)
    t = jnp.maximum(t + b1_ref[...], 0.0)
    ff = jnp.dot(t, w2_ref[...], preferred_element_type=jnp.float32)
    h = h + ff + b2_ref[...]
    out_ref[...] = g2_ref[...] * (h * c) + be2_ref[...]


def _final_kernel(x, xs, a0, a1, d0, d1, b_gcn, g1, be1, w1, b1, w2, b2,
                  g2, be2):
    row = lambda i: (i, 0)
    full = lambda shape: pl.BlockSpec(shape, lambda i: (0, 0))
    return pl.pallas_call(
        _final_body,
        out_shape=jax.ShapeDtypeStruct((N_NODES, DIM), jnp.float32),
        grid=(N_NODES // ROW_BLK,),
        in_specs=[
            pl.BlockSpec((ROW_BLK, DIM), row),
            pl.BlockSpec((ROW_BLK, DIM), row),
            pl.BlockSpec((ROW_BLK, DIM), row),
            pl.BlockSpec((ROW_BLK, DIM), row),
            pl.BlockSpec((ROW_BLK, 1), row),
            pl.BlockSpec((ROW_BLK, 1), row),
            full((1, DIM)),
            full((1, DIM)),
            full((1, DIM)),
            full((DIM, 2 * DIM)),
            full((1, 2 * DIM)),
            full((2 * DIM, DIM)),
            full((1, DIM)),
            full((1, DIM)),
            full((1, DIM)),
        ],
        out_specs=pl.BlockSpec((ROW_BLK, DIM), row),
    )(x, xs, a0, a1, d0, d1, b_gcn, g1, be1, w1, b1, w2, b2, g2, be2)


def kernel(x, edge_index, edge_attr, W_gcn, b_gcn, bn1_g, bn1_b, W1, b1,
           W2, b2, bn2_g, bn2_b):
    del edge_attr
    src = edge_index[0].astype(jnp.int32)
    dst = edge_index[1].astype(jnp.int32)
    npad = E_PAD - N_EDGES
    src = jnp.concatenate([src, jnp.zeros((npad,), jnp.int32)])
    dst = jnp.concatenate([dst, jnp.full((npad,), N_NODES, jnp.int32)])

    deg = _deg_kernel(dst).reshape(NC, N_PAD)
    d0 = deg[0, :N_NODES, None]
    d1 = deg[1, :N_NODES, None]
    xs = _scale_kernel(x, W_gcn, d0, d1)
    acc = _gather_scatter_kernel(xs, src, dst).reshape(NC, N_PAD, DIM)
    return _final_kernel(
        x, xs, acc[0, :N_NODES], acc[1, :N_NODES], d0, d1,
        b_gcn[None, :], bn1_g[None, :], bn1_b[None, :],
        W1, b1[None, :], W2, b2[None, :], bn2_g[None, :], bn2_b[None, :])
